# baseline (speedup 1.0000x reference)
import jax
import jax.numpy as jnp
from jax.experimental import pallas as pl
from jax.experimental.pallas import tpu as pltpu

_TIME = 10000
_OUT = 64
_IN = 64
_LANES = 128
_NCH = 8
_CO = _OUT // _NCH


def _body(t_ref, mat_hbm, out_ref, buf, sem):
    ti = t_ref[0].astype(jnp.int32)
    base = pl.multiple_of((ti // _LANES) * _LANES, _LANES)
    j = ti % _LANES
    for k in range(_NCH):
        pltpu.make_async_copy(
            mat_hbm.at[pl.ds(k * _CO, _CO), :, pl.ds(base, _LANES)],
            buf.at[k],
            sem.at[k],
        ).start()
    lane = jax.lax.broadcasted_iota(jnp.int32, (_CO, _IN, _LANES), 2)
    for k in range(_NCH):
        pltpu.make_async_copy(
            mat_hbm.at[pl.ds(k * _CO, _CO), :, pl.ds(base, _LANES)],
            buf.at[k],
            sem.at[k],
        ).wait()
        out_ref[pl.ds(k * _CO, _CO), :] = jnp.sum(
            jnp.where(lane == j, buf[k], 0.0), axis=2
        )


@jax.jit
def _lookup(mat_t, t1):
    return pl.pallas_call(
        _body,
        in_specs=[
            pl.BlockSpec(memory_space=pltpu.SMEM),
            pl.BlockSpec(memory_space=pl.ANY),
        ],
        out_specs=pl.BlockSpec(memory_space=pltpu.VMEM),
        out_shape=jax.ShapeDtypeStruct((_OUT, _IN), jnp.float32),
        scratch_shapes=[
            pltpu.VMEM((_NCH, _CO, _IN, _LANES), jnp.float32),
            pltpu.SemaphoreType.DMA((_NCH,)),
        ],
    )(t1, mat_t)


def kernel(matrix, t):
    mat_t = jnp.transpose(matrix, (1, 2, 0))
    return _lookup(mat_t, t.reshape(1))


# speedup vs baseline: 1.0655x; 1.0655x over previous
import jax
import jax.numpy as jnp
from jax.experimental import pallas as pl
from jax.experimental.pallas import tpu as pltpu

_TIME = 10000
_OUT = 64
_IN = 64
_LANES = 128
_NCH = 2
_CO = _OUT // _NCH


def _body(t_ref, mat_hbm, out_ref, buf, sem):
    ti = t_ref[0].astype(jnp.int32)
    base = pl.multiple_of((ti // _LANES) * _LANES, _LANES)
    j = ti % _LANES
    for k in range(_NCH):
        pltpu.make_async_copy(
            mat_hbm.at[pl.ds(k * _CO, _CO), :, pl.ds(base, _LANES)],
            buf.at[k],
            sem.at[k],
        ).start()
    lane = jax.lax.broadcasted_iota(jnp.int32, (_CO, _IN, _LANES), 2)
    for k in range(_NCH):
        pltpu.make_async_copy(
            mat_hbm.at[pl.ds(k * _CO, _CO), :, pl.ds(base, _LANES)],
            buf.at[k],
            sem.at[k],
        ).wait()
        out_ref[pl.ds(k * _CO, _CO), :] = jnp.sum(
            jnp.where(lane == j, buf[k], 0.0), axis=2
        )


@jax.jit
def _lookup(mat_t, t1):
    return pl.pallas_call(
        _body,
        in_specs=[
            pl.BlockSpec(memory_space=pltpu.SMEM),
            pl.BlockSpec(memory_space=pl.ANY),
        ],
        out_specs=pl.BlockSpec(memory_space=pltpu.VMEM),
        out_shape=jax.ShapeDtypeStruct((_OUT, _IN), jnp.float32),
        scratch_shapes=[
            pltpu.VMEM((_NCH, _CO, _IN, _LANES), jnp.float32),
            pltpu.SemaphoreType.DMA((_NCH,)),
        ],
    )(t1, mat_t)


def kernel(matrix, t):
    mat_t = jnp.transpose(matrix, (1, 2, 0))
    return _lookup(mat_t, t.reshape(1))


# speedup vs baseline: 1.0686x; 1.0029x over previous
import jax
import jax.numpy as jnp
from jax.experimental import pallas as pl
from jax.experimental.pallas import tpu as pltpu

_TIME = 10000
_OUT = 64
_IN = 64
_LANES = 128
_NCH = 4
_CO = _OUT // _NCH


def _body(t_ref, mat_hbm, out_ref, buf, sem):
    ti = t_ref[0].astype(jnp.int32)
    base = pl.multiple_of((ti // _LANES) * _LANES, _LANES)
    j = ti % _LANES
    for k in range(_NCH):
        pltpu.make_async_copy(
            mat_hbm.at[pl.ds(k * _CO, _CO), :, pl.ds(base, _LANES)],
            buf.at[k],
            sem.at[k],
        ).start()
    lane = jax.lax.broadcasted_iota(jnp.int32, (_CO, _IN, _LANES), 2)
    for k in range(_NCH):
        pltpu.make_async_copy(
            mat_hbm.at[pl.ds(k * _CO, _CO), :, pl.ds(base, _LANES)],
            buf.at[k],
            sem.at[k],
        ).wait()
        out_ref[pl.ds(k * _CO, _CO), :] = jnp.sum(
            jnp.where(lane == j, buf[k], 0.0), axis=2
        )


@jax.jit
def _lookup(mat_t, t1):
    return pl.pallas_call(
        _body,
        in_specs=[
            pl.BlockSpec(memory_space=pltpu.SMEM),
            pl.BlockSpec(memory_space=pl.ANY),
        ],
        out_specs=pl.BlockSpec(memory_space=pltpu.VMEM),
        out_shape=jax.ShapeDtypeStruct((_OUT, _IN), jnp.float32),
        scratch_shapes=[
            pltpu.VMEM((_NCH, _CO, _IN, _LANES), jnp.float32),
            pltpu.SemaphoreType.DMA((_NCH,)),
        ],
    )(t1, mat_t)


def kernel(matrix, t):
    mat_t = jnp.transpose(matrix, (1, 2, 0))
    return _lookup(mat_t, t.reshape(1))
